# baseline (speedup 1.0000x reference)
import functools

import jax
import jax.numpy as jnp
from jax import lax
from jax.experimental import pallas as pl
from jax.experimental.pallas import tpu as pltpu
from jax.experimental.pallas import tpu_sc as plsc

B, L, T, KH, DH = 16, 2, 2048, 8, 64
HD = KH * DH
NROWS = B * L * T * 2

_mesh = plsc.VectorSubcoreMesh(core_axis_name="c", subcore_axis_name="s")


@functools.partial(
    pl.kernel,
    mesh=_mesh,
    scratch_types=[
        pltpu.VMEM((2 * B,), jnp.int32),
        pltpu.VMEM((2 * B, HD), jnp.float32),
        pltpu.SemaphoreType.DMA,
    ],
)
def _sc_scatter(rows_hbm, kvr_hbm, buf_ref, idxv, datv, sem):
    wid = lax.axis_index("s") * 2 + lax.axis_index("c")

    @pl.when(wid == 0)
    def _():
        pltpu.sync_copy(rows_hbm, idxv)
        pltpu.sync_copy(kvr_hbm, datv)
        scat = pltpu.make_async_copy(datv, buf_ref.at[idxv], sem)
        scat.start()
        scat.wait()


@jax.jit
def _run(rows, kvr, buf2):
    ref = jax.new_ref(buf2)
    _sc_scatter(rows, kvr, ref)
    return ref[...]


def kernel(buf, k_step, v_step, layer, idx):
    layer = jnp.clip(jnp.asarray(layer, jnp.int32), 0, L - 1)
    idx = jnp.clip(jnp.asarray(idx, jnp.int32), 0, T - 1)
    k2 = k_step.reshape(B, HD)
    v2 = v_step.reshape(B, HD)
    kvr = jnp.stack([k2, v2], axis=1).reshape(2 * B, HD)
    b_ar = jnp.arange(B, dtype=jnp.int32)
    base = ((b_ar * L + layer) * T + idx) * 2
    rows = (base[:, None] + jnp.arange(2, dtype=jnp.int32)[None, :]).reshape(-1)
    out2 = _run(rows, kvr, buf.reshape(NROWS, HD))
    return out2.reshape(B, L, T, 2, KH, DH)


# speedup vs baseline: 1.6694x; 1.6694x over previous
import jax
import jax.numpy as jnp
from jax.experimental import pallas as pl
from jax.experimental.pallas import tpu as pltpu

B, L, T, KH, DH = 16, 2, 2048, 8, 64
HD = KH * DH


def _body(layer_ref, idx_ref, k_ref, v_ref, buf_any, out_any, sem):
    del buf_any
    layer = layer_ref[0]
    idx = idx_ref[0]
    for b in range(B):
        pltpu.make_async_copy(
            k_ref.at[b], out_any.at[b * L + layer, idx, 0], sem
        ).start()
        pltpu.make_async_copy(
            v_ref.at[b], out_any.at[b * L + layer, idx, 1], sem
        ).start()
    for b in range(B):
        pltpu.make_async_copy(
            k_ref.at[b], out_any.at[b * L + layer, idx, 0], sem
        ).wait()
        pltpu.make_async_copy(
            v_ref.at[b], out_any.at[b * L + layer, idx, 1], sem
        ).wait()


@jax.jit
def _run(layer_s, idx_s, k2, v2, buf4):
    return pl.pallas_call(
        _body,
        in_specs=[
            pl.BlockSpec(memory_space=pltpu.SMEM),
            pl.BlockSpec(memory_space=pltpu.SMEM),
            pl.BlockSpec(memory_space=pltpu.VMEM),
            pl.BlockSpec(memory_space=pltpu.VMEM),
            pl.BlockSpec(memory_space=pl.ANY),
        ],
        out_specs=pl.BlockSpec(memory_space=pl.ANY),
        out_shape=jax.ShapeDtypeStruct((B * L, T, 2, HD), jnp.float32),
        scratch_shapes=[pltpu.SemaphoreType.DMA],
        input_output_aliases={4: 0},
    )(layer_s, idx_s, k2, v2, buf4)


def kernel(buf, k_step, v_step, layer, idx):
    layer = jnp.clip(jnp.asarray(layer, jnp.int32), 0, L - 1)
    idx = jnp.clip(jnp.asarray(idx, jnp.int32), 0, T - 1)
    k2 = k_step.reshape(B, HD)
    v2 = v_step.reshape(B, HD)
    out4 = _run(
        layer.reshape(1), idx.reshape(1), k2, v2, buf.reshape(B * L, T, 2, HD)
    )
    return out4.reshape(B, L, T, 2, KH, DH)
